# Initial kernel scaffold; baseline (speedup 1.0000x reference)
import jax
import jax.numpy as jnp
from jax import lax
from jax.experimental import pallas as pl
from jax.experimental.pallas import tpu as pltpu
from jax.experimental.pallas import tpu_sc as plsc

_N = 50000
_E = 800000
_H = 64
_G = 128
_NP = 50176
_RS = _NP // 16
_RZ = _RS // 8
_EP = 802816
_CS = _EP // 16 // 128
_CD = _EP // 32 // 128
_BLK = 1024
_GRID = _NP // _BLK


def _deg_body(dst_hbm, ones_hbm, z_hbm, out_hbm, dst_v, ones_v, zb_v, dsh):
    c = lax.axis_index("c")
    s = lax.axis_index("s")
    pltpu.sync_copy(z_hbm, zb_v)
    for k in range(8):
        pltpu.sync_copy(zb_v, dsh.at[pl.ds(s * _RS + k * _RZ, _RZ)])
    pltpu.sync_copy(ones_hbm, ones_v)
    pltpu.sync_copy(dst_hbm.at[c * 16 + s], dst_v)
    plsc.subcore_barrier()

    def body(j, carry):
        pltpu.sync_copy(ones_v, dsh.at[dst_v.at[j]], add=True)
        return carry

    lax.fori_loop(0, _CD, body, 0)
    plsc.subcore_barrier()
    pltpu.sync_copy(dsh.at[pl.ds(s * _RS, _RS)],
                    out_hbm.at[pl.ds(c * _NP + s * _RS, _RS)])


def _make_deg():
    mesh = plsc.VectorSubcoreMesh(core_axis_name="c", subcore_axis_name="s",
                                  num_cores=2, num_subcores=16)
    return pl.kernel(
        _deg_body,
        out_type=jax.ShapeDtypeStruct((2 * _NP, 16), jnp.float32),
        mesh=mesh,
        scratch_types=[
            pltpu.VMEM((_CD, 128), jnp.int32),
            pltpu.VMEM((128, 16), jnp.float32),
            pltpu.VMEM((_RZ, 16), jnp.float32),
            pltpu.VMEM_SHARED((_NP, 16), jnp.float32),
        ],
    )


def _spmm_body(yv_hbm, src_hbm, dst_hbm, z_hbm, out_hbm,
               src_v, dst_v, rows_v, zb_v, sem, ssh):
    c = lax.axis_index("c")
    s = lax.axis_index("s")
    pltpu.sync_copy(z_hbm, zb_v)
    for k in range(8):
        pltpu.sync_copy(zb_v, ssh.at[pl.ds(s * _RS + k * _RZ, _RZ)])
    pltpu.sync_copy(src_hbm.at[c, s], src_v)
    pltpu.sync_copy(dst_hbm.at[s], dst_v)
    plsc.subcore_barrier()

    def body(j, carry):
        pltpu.async_copy(yv_hbm.at[src_v.at[j]], rows_v, sem).wait()
        pltpu.sync_copy(rows_v, ssh.at[dst_v.at[j]], add=True)
        return carry

    lax.fori_loop(0, _CS, body, 0)
    plsc.subcore_barrier()
    pltpu.sync_copy(ssh.at[pl.ds(s * _RS, _RS)],
                    out_hbm.at[pl.ds(c * _NP + s * _RS, _RS)])


def _make_spmm():
    mesh = plsc.VectorSubcoreMesh(core_axis_name="c", subcore_axis_name="s",
                                  num_cores=2, num_subcores=16)
    return pl.kernel(
        _spmm_body,
        out_type=jax.ShapeDtypeStruct((2 * _NP, 32), jnp.float32),
        mesh=mesh,
        scratch_types=[
            pltpu.VMEM((_CS, 128), jnp.int32),
            pltpu.VMEM((_CS, 128), jnp.int32),
            pltpu.VMEM((128, 32), jnp.float32),
            pltpu.VMEM((_RZ, 32), jnp.float32),
            pltpu.SemaphoreType.DMA,
            pltpu.VMEM_SHARED((_NP, 32), jnp.float32),
        ],
    )


def _k_in_body(x_ref, da_ref, db_ref, fw_ref, fb_ref, w1_ref, o_ref):
    i = pl.program_id(0)
    dinv = lax.rsqrt(da_ref[:, 0:1] + db_ref[:, 0:1] + 1.0)
    h0 = jnp.maximum(x_ref[...] * fw_ref[...] + fb_ref[...], 0.0)
    y = dinv * jnp.dot(h0, w1_ref[...], preferred_element_type=jnp.float32)
    rows = i * _BLK + lax.broadcasted_iota(jnp.int32, (_BLK, 1), 0)
    o_ref[...] = jnp.where(rows < _N, y, 0.0)


def _k_mid_body(sa_ref, sb_ref, y_ref, da_ref, db_ref, w_ref, b_ref, o_ref):
    i = pl.program_id(0)
    dinv = lax.rsqrt(da_ref[:, 0:1] + db_ref[:, 0:1] + 1.0)
    agg = jnp.concatenate([sa_ref[...], sb_ref[...]], axis=1)
    h = jnp.maximum(dinv * (agg + y_ref[...]) + b_ref[...], 0.0)
    y2 = dinv * jnp.dot(h, w_ref[...], preferred_element_type=jnp.float32)
    rows = i * _BLK + lax.broadcasted_iota(jnp.int32, (_BLK, 1), 0)
    o_ref[...] = jnp.where(rows < _N, y2, 0.0)


def _k_out_body(sa_ref, sb_ref, y_ref, da_ref, db_ref, b_ref, bat_ref,
                o_ref, acc, cnt):
    i = pl.program_id(0)
    dinv = lax.rsqrt(da_ref[:, 0:1] + db_ref[:, 0:1] + 1.0)
    agg = jnp.concatenate([sa_ref[...], sb_ref[...]], axis=1)
    h = jnp.maximum(dinv * (agg + y_ref[...]) + b_ref[...], 0.0)
    onehot = (bat_ref[...] == lax.broadcasted_iota(jnp.int32, (1, _G), 1)
              ).astype(jnp.float32)

    @pl.when(i == 0)
    def _():
        acc[...] = jnp.zeros_like(acc)
        cnt[...] = jnp.zeros_like(cnt)

    dn = (((0,), (0,)), ((), ()))
    acc[...] += lax.dot_general(onehot, h, dn,
                                preferred_element_type=jnp.float32)
    cnt[...] += lax.dot_general(onehot, jnp.ones((_BLK, 1), jnp.float32), dn,
                                preferred_element_type=jnp.float32)

    @pl.when(i == _GRID - 1)
    def _():
        o_ref[...] = acc[...] / jnp.maximum(cnt[...], 1.0)


def _node_spec(w):
    return pl.BlockSpec((_BLK, w), lambda i: (i, 0))


def _hi_spec(w):
    return pl.BlockSpec((_BLK, w), lambda i: (i + _GRID, 0))


def _full_spec(shape):
    return pl.BlockSpec(shape, lambda i: tuple(0 for _ in shape))


def _make_k_in():
    return pl.pallas_call(
        _k_in_body,
        grid=(_GRID,),
        in_specs=[_node_spec(1), _node_spec(16), _hi_spec(16),
                  _full_spec((1, _H)), _full_spec((1, _H)),
                  _full_spec((_H, _H))],
        out_specs=_node_spec(_H),
        out_shape=jax.ShapeDtypeStruct((_NP, _H), jnp.float32),
    )


def _make_k_mid():
    return pl.pallas_call(
        _k_mid_body,
        grid=(_GRID,),
        in_specs=[_node_spec(32), _hi_spec(32), _node_spec(_H),
                  _node_spec(16), _hi_spec(16),
                  _full_spec((_H, _H)), _full_spec((1, _H))],
        out_specs=_node_spec(_H),
        out_shape=jax.ShapeDtypeStruct((_NP, _H), jnp.float32),
    )


def _make_k_out():
    return pl.pallas_call(
        _k_out_body,
        grid=(_GRID,),
        in_specs=[_node_spec(32), _hi_spec(32), _node_spec(_H),
                  _node_spec(16), _hi_spec(16),
                  _full_spec((1, _H)), _node_spec(1)],
        out_specs=_full_spec((_G, _H)),
        out_shape=jax.ShapeDtypeStruct((_G, _H), jnp.float32),
        scratch_shapes=[pltpu.VMEM((_G, _H), jnp.float32),
                        pltpu.VMEM((_G, 1), jnp.float32)],
    )


def kernel(x, edge_index, batch, fc_W, fc_b, W1, b1, W2, b2):
    src = edge_index[0]
    dst = edge_index[1]
    pad_e = _EP - _E
    srcp = jnp.pad(src, (0, pad_e), constant_values=_N)
    dstp = jnp.pad(dst, (0, pad_e), constant_values=_N)
    src2 = jnp.stack([2 * srcp, 2 * srcp + 1]).reshape(2, 16, _CS, 128)
    dst16 = dstp.reshape(16, _CS, 128)
    dst32 = dstp.reshape(32, _CD, 128)
    ones16 = jnp.ones((128, 16), jnp.float32)
    z16 = jnp.zeros((_RZ, 16), jnp.float32)
    z32 = jnp.zeros((_RZ, 32), jnp.float32)
    xp = jnp.pad(x, ((0, _NP - _N), (0, 0)))
    batp = jnp.pad(batch, (0, _NP - _N), constant_values=_G).reshape(_NP, 1)
    fw = fc_W.reshape(1, _H)
    fb = fc_b.reshape(1, _H)
    b1r = b1.reshape(1, _H)
    b2r = b2.reshape(1, _H)

    degf = _make_deg()(dst32, ones16, z16)
    y1 = _make_k_in()(xp, degf, degf, fw, fb, W1)
    s1 = _make_spmm()(y1.reshape(2 * _NP, 32), src2, dst16, z32)
    y2 = _make_k_mid()(s1, s1, y1, degf, degf, W2, b1r)
    s2 = _make_spmm()(y2.reshape(2 * _NP, 32), src2, dst16, z32)
    out = _make_k_out()(s2, s2, y2, degf, degf, b2r, batp)
    return out


# speedup vs baseline: 7.0342x; 7.0342x over previous
import jax
import jax.numpy as jnp
from jax import lax
from jax.experimental import pallas as pl
from jax.experimental.pallas import tpu as pltpu
from jax.experimental.pallas import tpu_sc as plsc

_N = 50000
_E = 800000
_H = 64
_G = 128
_NP = 50176
_RS = _NP // 16
_RZ = _RS // 8
_EP = 802816
_CS = _EP // 16 // 128
_CD = _EP // 32 // 128
_W = 8
_NS = _H // _W
_BLK = 1024
_GRID = _NP // _BLK
_SC_PARAMS = pltpu.CompilerParams(use_tc_tiling_on_sc=False)
_MESH = dict(core_axis_name="c", subcore_axis_name="s",
             num_cores=2, num_subcores=16)


def _deg_body(dst_hbm, ones_hbm, z_hbm, out_hbm, dst_v, ones_v, dsh):
    c = lax.axis_index("c")
    s = lax.axis_index("s")
    pltpu.sync_copy(z_hbm, dsh.at[pl.ds(s * _RS, _RS)])
    pltpu.sync_copy(ones_hbm, ones_v)
    pltpu.sync_copy(dst_hbm.at[c * 16 + s], dst_v)
    plsc.subcore_barrier()

    def body(j, carry):
        pltpu.sync_copy(ones_v, dsh.at[dst_v.at[j]], add=True)
        return carry

    lax.fori_loop(0, _CD, body, 0)
    plsc.subcore_barrier()
    pltpu.sync_copy(dsh.at[pl.ds(s * _RS, _RS)],
                    out_hbm.at[pl.ds(c * _NP + s * _RS, _RS)])


def _make_deg():
    return pl.kernel(
        _deg_body,
        compiler_params=_SC_PARAMS,
        out_type=jax.ShapeDtypeStruct((2 * _NP,), jnp.float32),
        mesh=plsc.VectorSubcoreMesh(**_MESH),
        scratch_types=[
            pltpu.VMEM((_CD, 128), jnp.int32),
            pltpu.VMEM((128,), jnp.float32),
            pltpu.VMEM_SHARED((_NP,), jnp.float32),
        ],
    )


def _spmm_body(yv_hbm, src_hbm, dst_hbm, z_hbm, out_hbm,
               src_v, dst_v, rows_v, zb_v, sem, ssh):
    c = lax.axis_index("c")
    s = lax.axis_index("s")
    pltpu.sync_copy(z_hbm, zb_v)
    pltpu.sync_copy(dst_hbm.at[s], dst_v)
    for p in range(_NS // 2):
        for k in range(8):
            pltpu.sync_copy(zb_v, ssh.at[pl.ds(s * _RS + k * _RZ, _RZ)])
        pltpu.sync_copy(src_hbm.at[c, p, s], src_v)
        plsc.subcore_barrier()

        def body(j, carry):
            pltpu.async_copy(yv_hbm.at[src_v.at[j]], rows_v, sem).wait()
            pltpu.sync_copy(rows_v, ssh.at[dst_v.at[j]], add=True)
            return carry

        lax.fori_loop(0, _CS, body, 0)
        plsc.subcore_barrier()
        pltpu.sync_copy(
            ssh.at[pl.ds(s * _RS, _RS)],
            out_hbm.at[pl.ds(((_NS // 2) * c + p) * _NP + s * _RS, _RS)])


def _make_spmm():
    return pl.kernel(
        _spmm_body,
        compiler_params=_SC_PARAMS,
        out_type=jax.ShapeDtypeStruct((_NS * _NP, _W), jnp.float32),
        mesh=plsc.VectorSubcoreMesh(**_MESH),
        scratch_types=[
            pltpu.VMEM((_CS, 128), jnp.int32),
            pltpu.VMEM((_CS, 128), jnp.int32),
            pltpu.VMEM((128, _W), jnp.float32),
            pltpu.VMEM((_RZ, _W), jnp.float32),
            pltpu.SemaphoreType.DMA,
            pltpu.VMEM_SHARED((_NP, _W), jnp.float32),
        ],
    )


def _k_in_body(x_ref, fw_ref, fb_ref, o_ref):
    o_ref[...] = jnp.maximum(x_ref[...] * fw_ref[...] + fb_ref[...], 0.0)


def _k_y_body(h_ref, da_ref, db_ref, w_ref, o_ref):
    i = pl.program_id(0)
    dinv = lax.rsqrt(da_ref[...] + db_ref[...] + 1.0)
    y = dinv * jnp.dot(h_ref[...], w_ref[...],
                       preferred_element_type=jnp.float32)
    rows = i * _BLK + lax.broadcasted_iota(jnp.int32, (_BLK, 1), 0)
    o_ref[...] = jnp.where(rows < _N, y, 0.0)


def _k_mid_body(s0, s1, s2, s3, s4, s5, s6, s7, y_ref, da_ref, db_ref,
                w_ref, b_ref, o_ref):
    i = pl.program_id(0)
    dinv = lax.rsqrt(da_ref[...] + db_ref[...] + 1.0)
    agg = jnp.concatenate(
        [s0[...], s1[...], s2[...], s3[...],
         s4[...], s5[...], s6[...], s7[...]], axis=1)
    h = jnp.maximum(dinv * (agg + y_ref[...]) + b_ref[...], 0.0)
    y2 = dinv * jnp.dot(h, w_ref[...], preferred_element_type=jnp.float32)
    rows = i * _BLK + lax.broadcasted_iota(jnp.int32, (_BLK, 1), 0)
    o_ref[...] = jnp.where(rows < _N, y2, 0.0)


def _k_out_body(s0, s1, s2, s3, s4, s5, s6, s7, y_ref, da_ref, db_ref,
                b_ref, bat_ref, o_ref, acc, cnt):
    i = pl.program_id(0)
    dinv = lax.rsqrt(da_ref[...] + db_ref[...] + 1.0)
    agg = jnp.concatenate(
        [s0[...], s1[...], s2[...], s3[...],
         s4[...], s5[...], s6[...], s7[...]], axis=1)
    h = jnp.maximum(dinv * (agg + y_ref[...]) + b_ref[...], 0.0)
    onehot = (bat_ref[...] == lax.broadcasted_iota(jnp.int32, (1, _G), 1)
              ).astype(jnp.float32)

    @pl.when(i == 0)
    def _():
        acc[...] = jnp.zeros_like(acc)
        cnt[...] = jnp.zeros_like(cnt)

    dn = (((0,), (0,)), ((), ()))
    acc[...] += lax.dot_general(onehot, h, dn,
                                preferred_element_type=jnp.float32)
    cnt[...] += lax.dot_general(onehot, jnp.ones((_BLK, 1), jnp.float32), dn,
                                preferred_element_type=jnp.float32)

    @pl.when(i == _GRID - 1)
    def _():
        o_ref[...] = acc[...] / jnp.maximum(cnt[...], 1.0)


def _node_spec(w):
    return pl.BlockSpec((_BLK, w), lambda i: (i, 0))


def _q_spec(w, q):
    return pl.BlockSpec((_BLK, w), lambda i, q=q: (i + q * _GRID, 0))


def _full_spec(shape):
    return pl.BlockSpec(shape, lambda i: tuple(0 for _ in shape))


def _make_k_in():
    return pl.pallas_call(
        _k_in_body,
        grid=(_GRID,),
        in_specs=[_node_spec(1), _full_spec((1, _H)), _full_spec((1, _H))],
        out_specs=_node_spec(_H),
        out_shape=jax.ShapeDtypeStruct((_NP, _H), jnp.float32),
    )


def _make_k_y():
    return pl.pallas_call(
        _k_y_body,
        grid=(_GRID,),
        in_specs=[_node_spec(_H), _node_spec(1), _node_spec(1),
                  _full_spec((_H, _H))],
        out_specs=_node_spec(_H),
        out_shape=jax.ShapeDtypeStruct((_NP, _H), jnp.float32),
    )


def _make_k_mid():
    return pl.pallas_call(
        _k_mid_body,
        grid=(_GRID,),
        in_specs=[_q_spec(_W, q) for q in range(_NS)]
        + [_node_spec(_H), _node_spec(1), _node_spec(1),
           _full_spec((_H, _H)), _full_spec((1, _H))],
        out_specs=_node_spec(_H),
        out_shape=jax.ShapeDtypeStruct((_NP, _H), jnp.float32),
    )


def _make_k_out():
    return pl.pallas_call(
        _k_out_body,
        grid=(_GRID,),
        in_specs=[_q_spec(_W, q) for q in range(_NS)]
        + [_node_spec(_H), _node_spec(1), _node_spec(1),
           _full_spec((1, _H)), _node_spec(1)],
        out_specs=_full_spec((_G, _H)),
        out_shape=jax.ShapeDtypeStruct((_G, _H), jnp.float32),
        scratch_shapes=[pltpu.VMEM((_G, _H), jnp.float32),
                        pltpu.VMEM((_G, 1), jnp.float32)],
    )


def kernel(x, edge_index, batch, fc_W, fc_b, W1, b1, W2, b2):
    src = edge_index[0]
    dst = edge_index[1]
    pad_e = _EP - _E
    srcp = jnp.pad(src, (0, pad_e), constant_values=_N)
    dstp = jnp.pad(dst, (0, pad_e), constant_values=_N)
    src8 = jnp.stack([_NS * srcp + q for q in range(_NS)])
    src8 = src8.reshape(2, _NS // 2, 16, _CS, 128)
    dst16 = dstp.reshape(16, _CS, 128)
    dst32 = dstp.reshape(32, _CD, 128)
    ones1 = jnp.ones((128,), jnp.float32)
    z1 = jnp.zeros((_RS,), jnp.float32)
    zw = jnp.zeros((_RZ, _W), jnp.float32)
    xp = jnp.pad(x, ((0, _NP - _N), (0, 0)))
    batp = jnp.pad(batch, (0, _NP - _N), constant_values=_G).reshape(_NP, 1)
    fw = fc_W.reshape(1, _H)
    fb = fc_b.reshape(1, _H)
    b1r = b1.reshape(1, _H)
    b2r = b2.reshape(1, _H)

    spmm1 = _make_spmm()
    spmm2 = _make_spmm()
    degf = _make_deg()(dst32, ones1, z1)
    da = degf[:_NP].reshape(_NP, 1)
    db = degf[_NP:].reshape(_NP, 1)
    h0 = _make_k_in()(xp, fw, fb)
    y1 = _make_k_y()(h0, da, db, W1)
    s1 = spmm1(y1.reshape(_NS * _NP, _W), src8, dst16, zw)
    y2 = _make_k_mid()(*([s1] * _NS), y1, da, db, W2, b1r)
    s2 = spmm2(y2.reshape(_NS * _NP, _W), src8, dst16, zw)
    return _make_k_out()(*([s2] * _NS), y2, da, db, b2r, batp)


# speedup vs baseline: 7.4368x; 1.0572x over previous
import jax
import jax.numpy as jnp
from jax import lax
from jax.experimental import pallas as pl
from jax.experimental.pallas import tpu as pltpu
from jax.experimental.pallas import tpu_sc as plsc

_N = 50000
_E = 800000
_H = 64
_G = 128
_NP = 50176
_RS = _NP // 16
_RZ = _RS // 8
_EP = 802816
_CS = _EP // 16 // 128
_CD = _EP // 32 // 128
_W = 8
_NS = _H // _W
_BLK = 1024
_GRID = _NP // _BLK
_SC_PARAMS = pltpu.CompilerParams(use_tc_tiling_on_sc=False)
_MESH = dict(core_axis_name="c", subcore_axis_name="s",
             num_cores=2, num_subcores=16)


_HN = _NP // 2
_DT = _HN + 128
_DR = _DT // 16
_CDD = _EP // 16 // 128


def _deg_body(dst_hbm, ones_hbm, z_hbm, out_hbm, dst_v, ones_v, dsh):
    c = lax.axis_index("c")
    s = lax.axis_index("s")
    pltpu.sync_copy(z_hbm, dsh.at[pl.ds(s * _DR, _DR)])
    pltpu.sync_copy(ones_hbm, ones_v)
    pltpu.sync_copy(dst_hbm.at[c, s], dst_v)
    plsc.subcore_barrier()

    def body(j, carry):
        pltpu.sync_copy(ones_v, dsh.at[dst_v.at[j]], add=True)
        return carry

    lax.fori_loop(0, _CDD, body, 0)
    plsc.subcore_barrier()
    pltpu.sync_copy(dsh.at[pl.ds(s * _DR, _DR)],
                    out_hbm.at[pl.ds(c * _DT + s * _DR, _DR)])


def _make_deg():
    return pl.kernel(
        _deg_body,
        compiler_params=_SC_PARAMS,
        out_type=jax.ShapeDtypeStruct((2 * _DT,), jnp.float32),
        mesh=plsc.VectorSubcoreMesh(**_MESH),
        scratch_types=[
            pltpu.VMEM((_CDD, 128), jnp.int32),
            pltpu.VMEM((128,), jnp.float32),
            pltpu.VMEM_SHARED((_DT,), jnp.float32),
        ],
    )


def _spmm_body(kc, yv_hbm, src_hbm, dst_hbm, z_hbm, out_hbm,
               src_v, dst_v, rows_v, zb_v, sem, ssh):
    c = lax.axis_index("c")
    s = lax.axis_index("s")
    nc = _CS // kc
    pltpu.sync_copy(z_hbm, zb_v)
    pltpu.sync_copy(dst_hbm.at[s], dst_v)

    for p in range(_NS // 2):
        for k in range(8):
            pltpu.sync_copy(zb_v, ssh.at[pl.ds(s * _RS + k * _RZ, _RZ)])
        pltpu.sync_copy(src_hbm.at[c, p, s], src_v)
        plsc.subcore_barrier()

        def chunk(j, carry):
            pltpu.async_copy(yv_hbm.at[src_v.at[j]], rows_v, sem).wait()
            pltpu.sync_copy(rows_v, ssh.at[dst_v.at[j]], add=True)
            return carry

        lax.fori_loop(0, nc, chunk, 0)
        plsc.subcore_barrier()
        pltpu.sync_copy(
            ssh.at[pl.ds(s * _RS, _RS)],
            out_hbm.at[pl.ds(((_NS // 2) * c + p) * _NP + s * _RS, _RS)])


def _make_spmm(kc):
    def body(*refs):
        _spmm_body(kc, *refs)

    return pl.kernel(
        body,
        compiler_params=_SC_PARAMS,
        out_type=jax.ShapeDtypeStruct((_NS * _NP, _W), jnp.float32),
        mesh=plsc.VectorSubcoreMesh(**_MESH),
        scratch_types=[
            pltpu.VMEM((_CS // kc, kc * 128), jnp.int32),
            pltpu.VMEM((_CS // kc, kc * 128), jnp.int32),
            pltpu.VMEM((kc * 128, _W), jnp.float32),
            pltpu.VMEM((_RZ, _W), jnp.float32),
            pltpu.SemaphoreType.DMA,
            pltpu.VMEM_SHARED((_NP, _W), jnp.float32),
        ],
    )


def _k_in_body(x_ref, fw_ref, fb_ref, o_ref):
    o_ref[...] = jnp.maximum(x_ref[...] * fw_ref[...] + fb_ref[...], 0.0)


def _k_y_body(h_ref, da_ref, w_ref, o_ref):
    i = pl.program_id(0)
    dinv = lax.rsqrt(da_ref[...] + 1.0)
    y = dinv * jnp.dot(h_ref[...], w_ref[...],
                       preferred_element_type=jnp.float32)
    rows = i * _BLK + lax.broadcasted_iota(jnp.int32, (_BLK, 1), 0)
    o_ref[...] = jnp.where(rows < _N, y, 0.0)


def _k_mid_body(s0, s1, s2, s3, s4, s5, s6, s7, y_ref, da_ref,
                w_ref, b_ref, o_ref):
    i = pl.program_id(0)
    dinv = lax.rsqrt(da_ref[...] + 1.0)
    agg = jnp.concatenate(
        [s0[...], s1[...], s2[...], s3[...],
         s4[...], s5[...], s6[...], s7[...]], axis=1)
    h = jnp.maximum(dinv * (agg + y_ref[...]) + b_ref[...], 0.0)
    y2 = dinv * jnp.dot(h, w_ref[...], preferred_element_type=jnp.float32)
    rows = i * _BLK + lax.broadcasted_iota(jnp.int32, (_BLK, 1), 0)
    o_ref[...] = jnp.where(rows < _N, y2, 0.0)


def _k_out_body(s0, s1, s2, s3, s4, s5, s6, s7, y_ref, da_ref,
                b_ref, bat_ref, o_ref, acc, cnt):
    i = pl.program_id(0)
    dinv = lax.rsqrt(da_ref[...] + 1.0)
    agg = jnp.concatenate(
        [s0[...], s1[...], s2[...], s3[...],
         s4[...], s5[...], s6[...], s7[...]], axis=1)
    h = jnp.maximum(dinv * (agg + y_ref[...]) + b_ref[...], 0.0)
    onehot = (bat_ref[...] == lax.broadcasted_iota(jnp.int32, (1, _G), 1)
              ).astype(jnp.float32)

    @pl.when(i == 0)
    def _():
        acc[...] = jnp.zeros_like(acc)
        cnt[...] = jnp.zeros_like(cnt)

    dn = (((0,), (0,)), ((), ()))
    acc[...] += lax.dot_general(onehot, h, dn,
                                preferred_element_type=jnp.float32)
    cnt[...] += lax.dot_general(onehot, jnp.ones((_BLK, 1), jnp.float32), dn,
                                preferred_element_type=jnp.float32)

    @pl.when(i == _GRID - 1)
    def _():
        o_ref[...] = acc[...] / jnp.maximum(cnt[...], 1.0)


def _node_spec(w):
    return pl.BlockSpec((_BLK, w), lambda i: (i, 0))


def _q_spec(w, q):
    return pl.BlockSpec((_BLK, w), lambda i, q=q: (i + q * _GRID, 0))


def _full_spec(shape):
    return pl.BlockSpec(shape, lambda i: tuple(0 for _ in shape))


def _make_k_in():
    return pl.pallas_call(
        _k_in_body,
        grid=(_GRID,),
        in_specs=[_node_spec(1), _full_spec((1, _H)), _full_spec((1, _H))],
        out_specs=_node_spec(_H),
        out_shape=jax.ShapeDtypeStruct((_NP, _H), jnp.float32),
    )


def _make_k_y():
    return pl.pallas_call(
        _k_y_body,
        grid=(_GRID,),
        in_specs=[_node_spec(_H), _node_spec(1),
                  _full_spec((_H, _H))],
        out_specs=_node_spec(_H),
        out_shape=jax.ShapeDtypeStruct((_NP, _H), jnp.float32),
    )


def _make_k_mid():
    return pl.pallas_call(
        _k_mid_body,
        grid=(_GRID,),
        in_specs=[_q_spec(_W, q) for q in range(_NS)]
        + [_node_spec(_H), _node_spec(1),
           _full_spec((_H, _H)), _full_spec((1, _H))],
        out_specs=_node_spec(_H),
        out_shape=jax.ShapeDtypeStruct((_NP, _H), jnp.float32),
    )


def _make_k_out():
    return pl.pallas_call(
        _k_out_body,
        grid=(_GRID,),
        in_specs=[_q_spec(_W, q) for q in range(_NS)]
        + [_node_spec(_H), _node_spec(1),
           _full_spec((1, _H)), _node_spec(1)],
        out_specs=_full_spec((_G, _H)),
        out_shape=jax.ShapeDtypeStruct((_G, _H), jnp.float32),
        scratch_shapes=[pltpu.VMEM((_G, _H), jnp.float32),
                        pltpu.VMEM((_G, 1), jnp.float32)],
    )


def kernel(x, edge_index, batch, fc_W, fc_b, W1, b1, W2, b2):
    src = edge_index[0]
    dst = edge_index[1]
    pad_e = _EP - _E
    srcp = jnp.pad(src, (0, pad_e), constant_values=_N)
    dstp = jnp.pad(dst, (0, pad_e), constant_values=_N)
    src8 = jnp.stack([_NS * srcp + q for q in range(_NS)])
    src8a = src8.reshape(2, _NS // 2, 16, _CS // 2, 256)
    src8b = src8.reshape(2, _NS // 2, 16, _CS, 128)
    dst16a = dstp.reshape(16, _CS // 2, 256)
    dst16b = dstp.reshape(16, _CS, 128)
    dd0 = jnp.where(dstp < _HN, dstp, _HN)
    dd1 = jnp.where(dstp >= _HN, dstp - _HN, _HN)
    dd1 = jnp.where(dd1 > _HN, _HN, dd1)
    ddc = jnp.stack([dd0, dd1]).reshape(2, 16, _CDD, 128)
    ones1 = jnp.ones((128,), jnp.float32)
    z1 = jnp.zeros((_DR,), jnp.float32)
    zw = jnp.zeros((_RZ, _W), jnp.float32)
    xp = jnp.pad(x, ((0, _NP - _N), (0, 0)))
    batp = jnp.pad(batch, (0, _NP - _N), constant_values=_G).reshape(_NP, 1)
    fw = fc_W.reshape(1, _H)
    fb = fc_b.reshape(1, _H)
    b1r = b1.reshape(1, _H)
    b2r = b2.reshape(1, _H)

    spmm1 = _make_spmm(2)
    spmm2 = _make_spmm(1)
    degf = _make_deg()(ddc, ones1, z1)
    da = jnp.concatenate(
        [degf[:_HN], degf[_DT:_DT + _HN]]).reshape(_NP, 1)
    h0 = _make_k_in()(xp, fw, fb)
    y1 = _make_k_y()(h0, da, W1)
    s1 = spmm1(y1.reshape(_NS * _NP, _W), src8a, dst16a, zw)
    y2 = _make_k_mid()(*([s1] * _NS), y1, da, W2, b1r)
    s2 = spmm2(y2.reshape(_NS * _NP, _W), src8b, dst16b, zw)
    return _make_k_out()(*([s2] * _NS), y2, da, b2r, batp)


# speedup vs baseline: 8.6479x; 1.1629x over previous
import jax
import jax.numpy as jnp
from jax import lax
from jax.experimental import pallas as pl
from jax.experimental.pallas import tpu as pltpu
from jax.experimental.pallas import tpu_sc as plsc

_N = 50000
_E = 800000
_H = 64
_G = 128
_NP = 50176
_RS = _NP // 16
_RZ = _RS // 8
_EP = 802816
_CS = _EP // 16 // 128
_CD = _EP // 32 // 128
_W = 8
_NS = _H // _W
_BLK = 1024
_GRID = _NP // _BLK
_SC_PARAMS = pltpu.CompilerParams(use_tc_tiling_on_sc=False)
_MESH = dict(core_axis_name="c", subcore_axis_name="s",
             num_cores=2, num_subcores=16)


_HN = _NP // 2
_DT = _HN + 128
_DR = _DT // 16
_CDD = _EP // 16 // 128


def _deg_body(dst_hbm, ones_hbm, z_hbm, out_hbm, dst_v, ones_v, dsh):
    c = lax.axis_index("c")
    s = lax.axis_index("s")
    pltpu.sync_copy(z_hbm, dsh.at[pl.ds(s * _DR, _DR)])
    pltpu.sync_copy(ones_hbm, ones_v)
    pltpu.sync_copy(dst_hbm.at[c, s], dst_v)
    plsc.subcore_barrier()

    def body(j, carry):
        pltpu.sync_copy(ones_v, dsh.at[dst_v.at[j]], add=True)
        return carry

    lax.fori_loop(0, _CDD, body, 0)
    plsc.subcore_barrier()
    pltpu.sync_copy(dsh.at[pl.ds(s * _DR, _DR)],
                    out_hbm.at[pl.ds(c * _DT + s * _DR, _DR)])


def _make_deg():
    return pl.kernel(
        _deg_body,
        compiler_params=_SC_PARAMS,
        out_type=jax.ShapeDtypeStruct((2 * _DT,), jnp.float32),
        mesh=plsc.VectorSubcoreMesh(**_MESH),
        scratch_types=[
            pltpu.VMEM((_CDD, 128), jnp.int32),
            pltpu.VMEM((128,), jnp.float32),
            pltpu.VMEM_SHARED((_DT,), jnp.float32),
        ],
    )


def _spmm_body(kc, yv_hbm, src_hbm, dst_hbm, z_hbm, out_hbm,
               src_v, dst_v, rows_v, zb_v, sem, ssh):
    c = lax.axis_index("c")
    s = lax.axis_index("s")
    nc = _CS // kc
    pltpu.sync_copy(z_hbm, zb_v)
    pltpu.sync_copy(dst_hbm.at[s], dst_v)

    for p in range(_NS // 2):
        for k in range(8):
            pltpu.sync_copy(zb_v, ssh.at[pl.ds(s * _RS + k * _RZ, _RZ)])
        pltpu.sync_copy(src_hbm.at[c, p, s], src_v)
        plsc.subcore_barrier()

        def chunk(j, carry):
            pltpu.async_copy(yv_hbm.at[src_v.at[j]], rows_v, sem).wait()
            pltpu.sync_copy(rows_v, ssh.at[dst_v.at[j]], add=True)
            return carry

        lax.fori_loop(0, nc, chunk, 0)
        plsc.subcore_barrier()
        pltpu.sync_copy(
            ssh.at[pl.ds(s * _RS, _RS)],
            out_hbm.at[pl.ds(((_NS // 2) * c + p) * _NP + s * _RS, _RS)])


def _make_spmm(kc):
    def body(*refs):
        _spmm_body(kc, *refs)

    return pl.kernel(
        body,
        compiler_params=_SC_PARAMS,
        out_type=jax.ShapeDtypeStruct((_NS * _NP, _W), jnp.float32),
        mesh=plsc.VectorSubcoreMesh(**_MESH),
        scratch_types=[
            pltpu.VMEM((_CS // kc, kc * 128), jnp.int32),
            pltpu.VMEM((_CS // kc, kc * 128), jnp.int32),
            pltpu.VMEM((kc * 128, _W), jnp.float32),
            pltpu.VMEM((_RZ, _W), jnp.float32),
            pltpu.SemaphoreType.DMA,
            pltpu.VMEM_SHARED((_NP, _W), jnp.float32),
        ],
    )


def _k_in_body(x_ref, fw_ref, fb_ref, o_ref):
    o_ref[...] = jnp.maximum(x_ref[...] * fw_ref[...] + fb_ref[...], 0.0)


def _k_y_body(h_ref, da_ref, w_ref, o_ref):
    i = pl.program_id(0)
    dinv = lax.rsqrt(da_ref[...] + 1.0)
    y = dinv * jnp.dot(h_ref[...], w_ref[...],
                       preferred_element_type=jnp.float32)
    rows = i * _BLK + lax.broadcasted_iota(jnp.int32, (_BLK, 1), 0)
    o_ref[...] = jnp.where(rows < _N, y, 0.0)


def _k_mid_body(s0, s1, s2, s3, s4, s5, s6, s7, y_ref, da_ref,
                w_ref, b_ref, o_ref):
    i = pl.program_id(0)
    dinv = lax.rsqrt(da_ref[...] + 1.0)
    agg = jnp.concatenate(
        [s0[...], s1[...], s2[...], s3[...],
         s4[...], s5[...], s6[...], s7[...]], axis=1)
    h = jnp.maximum(dinv * (agg + y_ref[...]) + b_ref[...], 0.0)
    y2 = dinv * jnp.dot(h, w_ref[...], preferred_element_type=jnp.float32)
    rows = i * _BLK + lax.broadcasted_iota(jnp.int32, (_BLK, 1), 0)
    o_ref[...] = jnp.where(rows < _N, y2, 0.0)


def _k_out_body(s0, s1, s2, s3, s4, s5, s6, s7, y_ref, da_ref,
                b_ref, bat_ref, o_ref, acc, cnt):
    i = pl.program_id(0)
    dinv = lax.rsqrt(da_ref[...] + 1.0)
    agg = jnp.concatenate(
        [s0[...], s1[...], s2[...], s3[...],
         s4[...], s5[...], s6[...], s7[...]], axis=1)
    h = jnp.maximum(dinv * (agg + y_ref[...]) + b_ref[...], 0.0)
    onehot = (bat_ref[...] == lax.broadcasted_iota(jnp.int32, (1, _G), 1)
              ).astype(jnp.float32)

    @pl.when(i == 0)
    def _():
        acc[...] = jnp.zeros_like(acc)
        cnt[...] = jnp.zeros_like(cnt)

    dn = (((0,), (0,)), ((), ()))
    acc[...] += lax.dot_general(onehot, h, dn,
                                preferred_element_type=jnp.float32)
    cnt[...] += lax.dot_general(onehot, jnp.ones((_BLK, 1), jnp.float32), dn,
                                preferred_element_type=jnp.float32)

    @pl.when(i == _GRID - 1)
    def _():
        o_ref[...] = acc[...] / jnp.maximum(cnt[...], 1.0)


def _node_spec(w):
    return pl.BlockSpec((_BLK, w), lambda i: (i, 0))


def _q_spec(w, q):
    return pl.BlockSpec((_BLK, w), lambda i, q=q: (i + q * _GRID, 0))


def _full_spec(shape):
    return pl.BlockSpec(shape, lambda i: tuple(0 for _ in shape))


def _make_k_in():
    return pl.pallas_call(
        _k_in_body,
        grid=(_GRID,),
        in_specs=[_node_spec(1), _full_spec((1, _H)), _full_spec((1, _H))],
        out_specs=_node_spec(_H),
        out_shape=jax.ShapeDtypeStruct((_NP, _H), jnp.float32),
    )


def _make_k_y():
    return pl.pallas_call(
        _k_y_body,
        grid=(_GRID,),
        in_specs=[_node_spec(_H), _node_spec(1),
                  _full_spec((_H, _H))],
        out_specs=_node_spec(_H),
        out_shape=jax.ShapeDtypeStruct((_NP, _H), jnp.float32),
    )


def _make_k_mid():
    return pl.pallas_call(
        _k_mid_body,
        grid=(_GRID,),
        in_specs=[_q_spec(_W, q) for q in range(_NS)]
        + [_node_spec(_H), _node_spec(1),
           _full_spec((_H, _H)), _full_spec((1, _H))],
        out_specs=_node_spec(_H),
        out_shape=jax.ShapeDtypeStruct((_NP, _H), jnp.float32),
    )


def _make_k_out():
    return pl.pallas_call(
        _k_out_body,
        grid=(_GRID,),
        in_specs=[_q_spec(_W, q) for q in range(_NS)]
        + [_node_spec(_H), _node_spec(1),
           _full_spec((1, _H)), _node_spec(1)],
        out_specs=_full_spec((_G, _H)),
        out_shape=jax.ShapeDtypeStruct((_G, _H), jnp.float32),
        scratch_shapes=[pltpu.VMEM((_G, _H), jnp.float32),
                        pltpu.VMEM((_G, 1), jnp.float32)],
    )


def kernel(x, edge_index, batch, fc_W, fc_b, W1, b1, W2, b2):
    src = edge_index[0]
    dst = edge_index[1]
    pad_e = _EP - _E
    srcp = jnp.pad(src, (0, pad_e), constant_values=_N)
    dstp = jnp.pad(dst, (0, pad_e), constant_values=_N)
    src8 = jnp.stack([_NS * srcp + q for q in range(_NS)])
    src8a = src8.reshape(2, _NS // 2, 16, _CS // 2, 256)
    src8b = src8.reshape(2, _NS // 2, 16, _CS // 2, 256)
    dst16a = dstp.reshape(16, _CS // 2, 256)
    dst16b = dstp.reshape(16, _CS // 2, 256)
    dd0 = jnp.where(dstp < _HN, dstp, _HN)
    dd1 = jnp.where(dstp >= _HN, dstp - _HN, _HN)
    dd1 = jnp.where(dd1 > _HN, _HN, dd1)
    ddc = jnp.stack([dd0, dd1]).reshape(2, 16, _CDD, 128)
    ones1 = jnp.ones((128,), jnp.float32)
    z1 = jnp.zeros((_DR,), jnp.float32)
    zw = jnp.zeros((_RZ, _W), jnp.float32)
    xp = jnp.pad(x, ((0, _NP - _N), (0, 0)))
    batp = jnp.pad(batch, (0, _NP - _N), constant_values=_G).reshape(_NP, 1)
    fw = fc_W.reshape(1, _H)
    fb = fc_b.reshape(1, _H)
    b1r = b1.reshape(1, _H)
    b2r = b2.reshape(1, _H)

    spmm1 = _make_spmm(2)
    spmm2 = _make_spmm(2)
    degf = _make_deg()(ddc, ones1, z1)
    da = jnp.concatenate(
        [degf[:_HN], degf[_DT:_DT + _HN]]).reshape(_NP, 1)
    h0 = _make_k_in()(xp, fw, fb)
    y1 = _make_k_y()(h0, da, W1)
    s1 = spmm1(y1.reshape(_NS * _NP, _W), src8a, dst16a, zw)
    y2 = _make_k_mid()(*([s1] * _NS), y1, da, W2, b1r)
    s2 = spmm2(y2.reshape(_NS * _NP, _W), src8b, dst16b, zw)
    return _make_k_out()(*([s2] * _NS), y2, da, b2r, batp)
